# Initial kernel scaffold; baseline (speedup 1.0000x reference)
import jax
import jax.numpy as jnp
from jax.experimental import pallas as pl


def kernel(x, edge_index, W0, b0, W1, b1, W2, b2, W3, b3, W4, b4):
    raise NotImplementedError("write your pallas kernel here")


# speedup vs baseline: 4.8286x; 4.8286x over previous
import functools

import jax
import jax.numpy as jnp
from jax import lax
from jax.experimental import pallas as pl
from jax.experimental.pallas import tpu as pltpu
from jax.experimental.pallas import tpu_sc as plsc

N_NODES = 10000
D = 128
NC = 2
NS = 16
LANES = 16
NW = NC * NS

CHUNK = 80
NPAD = 10240
ROWS_PER_SUB = NPAD // NS
ZROWS = 128

_mesh = plsc.VectorSubcoreMesh(core_axis_name="c", subcore_axis_name="s")


def _worker_edge_base(n_edges):
    eps = n_edges // NW
    assert eps * NW == n_edges and eps % CHUNK == 0
    return eps


def _sc_degrees(src, dst):
    n_edges = src.shape[0]
    eps = _worker_edge_base(n_edges)
    n_chunks = eps // CHUNK

    @functools.partial(
        pl.kernel,
        out_type=jax.ShapeDtypeStruct((NC, 2, NPAD, LANES), jnp.float32),
        mesh=_mesh,
        scratch_types=[
            pltpu.VMEM((CHUNK,), jnp.int32),
            pltpu.VMEM((CHUNK, LANES), jnp.float32),
            pltpu.VMEM((ZROWS, LANES), jnp.float32),
            pltpu.VMEM_SHARED((NPAD, LANES), jnp.float32),
            pltpu.VMEM_SHARED((NPAD, LANES), jnp.float32),
        ],
        compiler_params=pltpu.CompilerParams(use_tc_tiling_on_sc=False),
    )
    def k(src_hbm, dst_hbm, out_hbm, idx_v, ones_v, zero_v, acc0, acc1):
        cid = lax.axis_index("c")
        sid = lax.axis_index("s")
        wid = cid * NS + sid

        def fill_ones(i, _):
            ones_v[i, :] = jnp.full((LANES,), 1.0, jnp.float32)
            return 0

        lax.fori_loop(0, CHUNK, fill_ones, 0)

        def fill_zero(i, _):
            zero_v[i, :] = jnp.zeros((LANES,), jnp.float32)
            return 0

        lax.fori_loop(0, ZROWS, fill_zero, 0)

        row0 = sid * ROWS_PER_SUB
        for t in range(ROWS_PER_SUB // ZROWS):
            pltpu.sync_copy(zero_v, acc0.at[pl.ds(row0 + t * ZROWS, ZROWS)])
            pltpu.sync_copy(zero_v, acc1.at[pl.ds(row0 + t * ZROWS, ZROWS)])
        plsc.subcore_barrier()

        ebase = wid * eps

        def step(c, _):
            off = ebase + c * CHUNK
            pltpu.sync_copy(src_hbm.at[pl.ds(off, CHUNK)], idx_v)
            pltpu.sync_copy(ones_v, acc0.at[idx_v], add=True)
            pltpu.sync_copy(dst_hbm.at[pl.ds(off, CHUNK)], idx_v)
            pltpu.sync_copy(ones_v, acc1.at[idx_v], add=True)
            return 0

        lax.fori_loop(0, n_chunks, step, 0)
        plsc.subcore_barrier()

        pltpu.sync_copy(acc0.at[pl.ds(row0, ROWS_PER_SUB)],
                        out_hbm.at[cid, 0, pl.ds(row0, ROWS_PER_SUB)])
        pltpu.sync_copy(acc1.at[pl.ds(row0, ROWS_PER_SUB)],
                        out_hbm.at[cid, 1, pl.ds(row0, ROWS_PER_SUB)])

    return k(src, dst)


def _sc_propagate(q, src, dst):
    n_edges = src.shape[0]
    eps = _worker_edge_base(n_edges)
    n_chunks = eps // CHUNK

    @functools.partial(
        pl.kernel,
        out_type=jax.ShapeDtypeStruct((NC, NPAD, D), jnp.float32),
        mesh=_mesh,
        scratch_types=[
            pltpu.VMEM((CHUNK,), jnp.int32),
            pltpu.VMEM((CHUNK,), jnp.int32),
            pltpu.VMEM((CHUNK, D), jnp.float32),
            pltpu.VMEM((ZROWS, D), jnp.float32),
            pltpu.VMEM_SHARED((NPAD, D), jnp.float32),
        ],
    )
    def k(q_hbm, src_hbm, dst_hbm, out_hbm, sidx_v, didx_v, rows_v, zero_v, acc):
        cid = lax.axis_index("c")
        sid = lax.axis_index("s")
        wid = cid * NS + sid

        def fill_zero(i, _):
            for j in range(D // LANES):
                zero_v[i, pl.ds(j * LANES, LANES)] = jnp.zeros((LANES,), jnp.float32)
            return 0

        lax.fori_loop(0, ZROWS, fill_zero, 0)

        row0 = sid * ROWS_PER_SUB
        for t in range(ROWS_PER_SUB // ZROWS):
            pltpu.sync_copy(zero_v, acc.at[pl.ds(row0 + t * ZROWS, ZROWS)])
        plsc.subcore_barrier()

        ebase = wid * eps

        def step(c, _):
            off = ebase + c * CHUNK
            pltpu.sync_copy(src_hbm.at[pl.ds(off, CHUNK)], sidx_v)
            pltpu.sync_copy(dst_hbm.at[pl.ds(off, CHUNK)], didx_v)
            pltpu.sync_copy(q_hbm.at[sidx_v], rows_v)
            pltpu.sync_copy(rows_v, acc.at[didx_v], add=True)
            return 0

        lax.fori_loop(0, n_chunks, step, 0)
        plsc.subcore_barrier()

        pltpu.sync_copy(acc.at[pl.ds(row0, ROWS_PER_SUB)],
                        out_hbm.at[cid, pl.ds(row0, ROWS_PER_SUB)])

    return k(q, src, dst)


_BLK = 2000


def _norms_from(dp):
    deg_out = dp[0, 0, :, 0] + dp[1, 0, :, 0]
    deg_in = dp[0, 1, :, 0] + dp[1, 1, :, 0]
    ns = lax.rsqrt(jnp.maximum(deg_out, 1.0))
    nd = lax.rsqrt(jnp.maximum(deg_in, 1.0))
    return ns, nd


def _tc_prep_body(x_ref, dp_ref, o_ref):
    ns, _ = _norms_from(dp_ref[...])
    o_ref[...] = x_ref[...] * ns[:, None]


def _tc_prep(x, degp):
    return pl.pallas_call(
        _tc_prep_body,
        out_shape=jax.ShapeDtypeStruct((N_NODES, D), jnp.float32),
        grid=(N_NODES // _BLK,),
        in_specs=[
            pl.BlockSpec((_BLK, D), lambda i: (i, 0)),
            pl.BlockSpec((NC, 2, _BLK, LANES), lambda i: (0, 0, i, 0)),
        ],
        out_specs=pl.BlockSpec((_BLK, D), lambda i: (i, 0)),
    )(x, degp)


def _tc_layer_body(last, p_ref, dp_ref, w_ref, b_ref, o_ref):
    ns, nd = _norms_from(dp_ref[...])
    s = (p_ref[0] + p_ref[1]) * nd[:, None]
    h = jnp.dot(s, w_ref[...], preferred_element_type=jnp.float32)
    h = jnp.maximum(h + b_ref[...], 0.0)
    if not last:
        h = h * ns[:, None]
    o_ref[...] = h


def _tc_layer(p, degp, w, b2d, last):
    return pl.pallas_call(
        functools.partial(_tc_layer_body, last),
        out_shape=jax.ShapeDtypeStruct((N_NODES, D), jnp.float32),
        grid=(N_NODES // _BLK,),
        in_specs=[
            pl.BlockSpec((NC, _BLK, D), lambda i: (0, i, 0)),
            pl.BlockSpec((NC, 2, _BLK, LANES), lambda i: (0, 0, i, 0)),
            pl.BlockSpec((D, D), lambda i: (0, 0)),
            pl.BlockSpec((1, D), lambda i: (0, 0)),
        ],
        out_specs=pl.BlockSpec((_BLK, D), lambda i: (i, 0)),
    )(p, degp, w, b2d)


def kernel(x, edge_index, W0, b0, W1, b1, W2, b2, W3, b3, W4, b4):
    ei = edge_index.astype(jnp.int32)
    src = ei[0]
    dst = ei[1]
    degp = _sc_degrees(src, dst)
    q = _tc_prep(x, degp)
    Ws = [W0, W1, W2, W3, W4]
    bs = [b0, b1, b2, b3, b4]
    for l in range(5):
        p = _sc_propagate(q, src, dst)
        q = _tc_layer(p, degp, Ws[l], bs[l].reshape(1, D), last=(l == 4))
    return q


# speedup vs baseline: 5.7127x; 1.1831x over previous
import functools

import jax
import jax.numpy as jnp
from jax import lax
from jax.experimental import pallas as pl
from jax.experimental.pallas import tpu as pltpu
from jax.experimental.pallas import tpu_sc as plsc

N_NODES = 10000
D = 128
NC = 2
NS = 16
LANES = 16
NW = NC * NS

CHUNK = 40
NPAD = 10240
ROWS_PER_SUB = NPAD // NS
ZROWS = 128

_mesh = plsc.VectorSubcoreMesh(core_axis_name="c", subcore_axis_name="s")


def _worker_edge_base(n_edges):
    eps = n_edges // NW
    assert eps * NW == n_edges and eps % CHUNK == 0
    return eps


def _sc_degrees(src, dst):
    n_edges = src.shape[0]
    eps = _worker_edge_base(n_edges)
    n_chunks = eps // CHUNK

    @functools.partial(
        pl.kernel,
        out_type=jax.ShapeDtypeStruct((NC, 2, NPAD, LANES), jnp.float32),
        mesh=_mesh,
        scratch_types=[
            pltpu.VMEM((CHUNK,), jnp.int32),
            pltpu.VMEM((CHUNK, LANES), jnp.float32),
            pltpu.VMEM((ZROWS, LANES), jnp.float32),
            pltpu.VMEM_SHARED((NPAD, LANES), jnp.float32),
            pltpu.VMEM_SHARED((NPAD, LANES), jnp.float32),
        ],
        compiler_params=pltpu.CompilerParams(use_tc_tiling_on_sc=False),
    )
    def k(src_hbm, dst_hbm, out_hbm, idx_v, ones_v, zero_v, acc0, acc1):
        cid = lax.axis_index("c")
        sid = lax.axis_index("s")
        wid = cid * NS + sid

        def fill_ones(i, _):
            ones_v[i, :] = jnp.full((LANES,), 1.0, jnp.float32)
            return 0

        lax.fori_loop(0, CHUNK, fill_ones, 0)

        def fill_zero(i, _):
            zero_v[i, :] = jnp.zeros((LANES,), jnp.float32)
            return 0

        lax.fori_loop(0, ZROWS, fill_zero, 0)

        row0 = sid * ROWS_PER_SUB
        for t in range(ROWS_PER_SUB // ZROWS):
            pltpu.sync_copy(zero_v, acc0.at[pl.ds(row0 + t * ZROWS, ZROWS)])
            pltpu.sync_copy(zero_v, acc1.at[pl.ds(row0 + t * ZROWS, ZROWS)])
        plsc.subcore_barrier()

        ebase = wid * eps

        def step(c, _):
            off = ebase + c * CHUNK
            pltpu.sync_copy(src_hbm.at[pl.ds(off, CHUNK)], idx_v)
            pltpu.sync_copy(ones_v, acc0.at[idx_v], add=True)
            pltpu.sync_copy(dst_hbm.at[pl.ds(off, CHUNK)], idx_v)
            pltpu.sync_copy(ones_v, acc1.at[idx_v], add=True)
            return 0

        lax.fori_loop(0, n_chunks, step, 0)
        plsc.subcore_barrier()

        pltpu.sync_copy(acc0.at[pl.ds(row0, ROWS_PER_SUB)],
                        out_hbm.at[cid, 0, pl.ds(row0, ROWS_PER_SUB)])
        pltpu.sync_copy(acc1.at[pl.ds(row0, ROWS_PER_SUB)],
                        out_hbm.at[cid, 1, pl.ds(row0, ROWS_PER_SUB)])

    return k(src, dst)


def _sc_propagate(q, src3, dst3):
    n_chunks = src3.shape[1]

    @functools.partial(
        pl.kernel,
        out_type=jax.ShapeDtypeStruct((NC, NPAD, D), jnp.float32),
        mesh=_mesh,
        scratch_types=[
            pltpu.VMEM((n_chunks, CHUNK), jnp.int32),
            pltpu.VMEM((n_chunks, CHUNK), jnp.int32),
            pltpu.VMEM((2, CHUNK, D), jnp.float32),
            pltpu.VMEM_SHARED((NPAD, D), jnp.float32),
            pltpu.SemaphoreType.DMA,
            pltpu.SemaphoreType.DMA,
        ],
        compiler_params=pltpu.CompilerParams(use_tc_tiling_on_sc=False),
    )
    def k(q_hbm, src_hbm, dst_hbm, out_hbm, sidx_v, didx_v, rows_v,
          acc, sem0, sem1):
        cid = lax.axis_index("c")
        sid = lax.axis_index("s")
        wid = cid * NS + sid
        sems = (sem0, sem1)

        def fill_zero(i, _):
            for j in range(D // LANES):
                rows_v[0, i, pl.ds(j * LANES, LANES)] = jnp.zeros((LANES,), jnp.float32)
            return 0

        lax.fori_loop(0, CHUNK, fill_zero, 0)

        pltpu.sync_copy(src_hbm.at[wid], sidx_v)
        pltpu.sync_copy(dst_hbm.at[wid], didx_v)

        row0 = sid * ROWS_PER_SUB
        for t in range(ROWS_PER_SUB // CHUNK):
            pltpu.sync_copy(rows_v.at[0], acc.at[pl.ds(row0 + t * CHUNK, CHUNK)])
        plsc.subcore_barrier()

        def gather(c, b):
            pltpu.async_copy(q_hbm.at[sidx_v.at[c]], rows_v.at[b], sems[b])

        def wait_gather(c, b):
            pltpu.make_async_copy(q_hbm.at[sidx_v.at[c]], rows_v.at[b],
                                  sems[b]).wait()

        def scatter(c, b):
            pltpu.sync_copy(rows_v.at[b], acc.at[didx_v.at[c]], add=True)

        gather(0, 0)

        def step(o, _):
            for b in range(2):
                c = 2 * o + b
                wait_gather(c, b)
                gather(c + 1, 1 - b)
                scatter(c, b)
            return 0

        npairs = (n_chunks - 1) // 2
        lax.fori_loop(0, npairs, step, 0)
        for c in range(2 * npairs, n_chunks):
            wait_gather(c, c % 2)
            if c + 1 < n_chunks:
                gather(c + 1, (c + 1) % 2)
            scatter(c, c % 2)
        plsc.subcore_barrier()

        pltpu.sync_copy(acc.at[pl.ds(row0, ROWS_PER_SUB)],
                        out_hbm.at[cid, pl.ds(row0, ROWS_PER_SUB)])

    return k(q, src3, dst3)


_BLK = 2000


def _norms_from(dp):
    deg_out = dp[0, 0, :, 0] + dp[1, 0, :, 0]
    deg_in = dp[0, 1, :, 0] + dp[1, 1, :, 0]
    ns = lax.rsqrt(jnp.maximum(deg_out, 1.0))
    nd = lax.rsqrt(jnp.maximum(deg_in, 1.0))
    return ns, nd


def _tc_prep_body(x_ref, dp_ref, o_ref):
    ns, _ = _norms_from(dp_ref[...])
    o_ref[...] = x_ref[...] * ns[:, None]


def _tc_prep(x, degp):
    return pl.pallas_call(
        _tc_prep_body,
        out_shape=jax.ShapeDtypeStruct((N_NODES, D), jnp.float32),
        grid=(N_NODES // _BLK,),
        in_specs=[
            pl.BlockSpec((_BLK, D), lambda i: (i, 0)),
            pl.BlockSpec((NC, 2, _BLK, LANES), lambda i: (0, 0, i, 0)),
        ],
        out_specs=pl.BlockSpec((_BLK, D), lambda i: (i, 0)),
    )(x, degp)


def _tc_layer_body(last, p_ref, dp_ref, w_ref, b_ref, o_ref):
    ns, nd = _norms_from(dp_ref[...])
    s = (p_ref[0] + p_ref[1]) * nd[:, None]
    h = jnp.dot(s, w_ref[...], preferred_element_type=jnp.float32)
    h = jnp.maximum(h + b_ref[...], 0.0)
    if not last:
        h = h * ns[:, None]
    o_ref[...] = h


def _tc_layer(p, degp, w, b2d, last):
    return pl.pallas_call(
        functools.partial(_tc_layer_body, last),
        out_shape=jax.ShapeDtypeStruct((N_NODES, D), jnp.float32),
        grid=(N_NODES // _BLK,),
        in_specs=[
            pl.BlockSpec((NC, _BLK, D), lambda i: (0, i, 0)),
            pl.BlockSpec((NC, 2, _BLK, LANES), lambda i: (0, 0, i, 0)),
            pl.BlockSpec((D, D), lambda i: (0, 0)),
            pl.BlockSpec((1, D), lambda i: (0, 0)),
        ],
        out_specs=pl.BlockSpec((_BLK, D), lambda i: (i, 0)),
    )(p, degp, w, b2d)


def kernel(x, edge_index, W0, b0, W1, b1, W2, b2, W3, b3, W4, b4):
    ei = edge_index.astype(jnp.int32)
    src = ei[0]
    dst = ei[1]
    src3 = src.reshape(NW, -1, CHUNK)
    dst3 = dst.reshape(NW, -1, CHUNK)
    degp = _sc_degrees(src, dst)
    q = _tc_prep(x, degp)
    Ws = [W0, W1, W2, W3, W4]
    bs = [b0, b1, b2, b3, b4]
    for l in range(5):
        p = _sc_propagate(q, src3, dst3)
        q = _tc_layer(p, degp, Ws[l], bs[l].reshape(1, D), last=(l == 4))
    return q


# speedup vs baseline: 10.4568x; 1.8305x over previous
import functools

import jax
import jax.numpy as jnp
from jax import lax
from jax.experimental import pallas as pl
from jax.experimental.pallas import tpu as pltpu
from jax.experimental.pallas import tpu_sc as plsc

N_NODES = 10000
D = 128
NC = 2
NS = 16
LANES = 16
NW = NC * NS

CHUNK = 40
NPAD = 10240
ROWS_PER_SUB = NPAD // NS
NB = 4

_mesh = plsc.VectorSubcoreMesh(core_axis_name="c", subcore_axis_name="s")
_sc_params = pltpu.CompilerParams(use_tc_tiling_on_sc=False)


def _sc_degrees(src3, dst3):
    n_chunks = src3.shape[1]

    @functools.partial(
        pl.kernel,
        out_type=jax.ShapeDtypeStruct((NC, 2, NPAD, LANES), jnp.float32),
        mesh=_mesh,
        scratch_types=[
            pltpu.VMEM((n_chunks, CHUNK), jnp.int32),
            pltpu.VMEM((n_chunks, CHUNK), jnp.int32),
            pltpu.VMEM((CHUNK, LANES), jnp.float32),
            pltpu.VMEM((ROWS_PER_SUB, LANES), jnp.float32),
            pltpu.VMEM_SHARED((NPAD, LANES), jnp.float32),
            pltpu.VMEM_SHARED((NPAD, LANES), jnp.float32),
            pltpu.SemaphoreType.DMA,
        ],
        compiler_params=_sc_params,
    )
    def k(src_hbm, dst_hbm, out_hbm, sidx_v, didx_v, ones_v, zero_v,
          acc0, acc1, ssem):
        cid = lax.axis_index("c")
        sid = lax.axis_index("s")
        wid = cid * NS + sid

        def fill_ones(i, _):
            ones_v[i, :] = jnp.full((LANES,), 1.0, jnp.float32)
            return 0

        lax.fori_loop(0, CHUNK, fill_ones, 0)

        def fill_zero(i, _):
            zero_v[i, :] = jnp.zeros((LANES,), jnp.float32)
            return 0

        lax.fori_loop(0, ROWS_PER_SUB, fill_zero, 0)

        pltpu.sync_copy(src_hbm.at[wid], sidx_v)
        pltpu.sync_copy(dst_hbm.at[wid], didx_v)

        row0 = sid * ROWS_PER_SUB
        pltpu.sync_copy(zero_v, acc0.at[pl.ds(row0, ROWS_PER_SUB)])
        pltpu.sync_copy(zero_v, acc1.at[pl.ds(row0, ROWS_PER_SUB)])
        plsc.subcore_barrier()

        def fire(c, _):
            pltpu.async_copy(ones_v, acc0.at[sidx_v.at[c]], ssem, add=True)
            pltpu.async_copy(ones_v, acc1.at[didx_v.at[c]], ssem, add=True)
            return 0

        lax.fori_loop(0, n_chunks, fire, 0)

        def drain(c, _):
            pltpu.make_async_copy(ones_v, acc0.at[sidx_v.at[c]], ssem).wait()
            pltpu.make_async_copy(ones_v, acc1.at[didx_v.at[c]], ssem).wait()
            return 0

        lax.fori_loop(0, n_chunks, drain, 0)
        plsc.subcore_barrier()

        pltpu.sync_copy(acc0.at[pl.ds(row0, ROWS_PER_SUB)],
                        out_hbm.at[cid, 0, pl.ds(row0, ROWS_PER_SUB)])
        pltpu.sync_copy(acc1.at[pl.ds(row0, ROWS_PER_SUB)],
                        out_hbm.at[cid, 1, pl.ds(row0, ROWS_PER_SUB)])

    return k(src3, dst3)


def _sc_propagate(q, src3, dst3):
    n_chunks = src3.shape[1]
    assert n_chunks >= 8

    @functools.partial(
        pl.kernel,
        out_type=jax.ShapeDtypeStruct((NC, NPAD, D), jnp.float32),
        mesh=_mesh,
        scratch_types=[
            pltpu.VMEM((n_chunks, CHUNK), jnp.int32),
            pltpu.VMEM((n_chunks, CHUNK), jnp.int32),
            pltpu.VMEM((NB, CHUNK, D), jnp.float32),
            pltpu.VMEM_SHARED((NPAD, D), jnp.float32),
            [pltpu.SemaphoreType.DMA] * NB,
            [pltpu.SemaphoreType.DMA] * NB,
        ],
        compiler_params=_sc_params,
    )
    def k(q_hbm, src_hbm, dst_hbm, out_hbm, sidx_v, didx_v, rows_v,
          acc, gsems, ssems):
        cid = lax.axis_index("c")
        sid = lax.axis_index("s")
        wid = cid * NS + sid

        def fill_zero(i, _):
            for j in range(D // LANES):
                rows_v[0, i, pl.ds(j * LANES, LANES)] = jnp.zeros(
                    (LANES,), jnp.float32)
            return 0

        lax.fori_loop(0, CHUNK, fill_zero, 0)

        pltpu.sync_copy(src_hbm.at[wid], sidx_v)
        pltpu.sync_copy(dst_hbm.at[wid], didx_v)

        row0 = sid * ROWS_PER_SUB
        for t in range(ROWS_PER_SUB // CHUNK):
            pltpu.sync_copy(rows_v.at[0], acc.at[pl.ds(row0 + t * CHUNK, CHUNK)])
        plsc.subcore_barrier()

        def gather(c, b):
            pltpu.async_copy(q_hbm.at[sidx_v.at[c]], rows_v.at[b], gsems[b])

        def wait_gather(c, b):
            pltpu.make_async_copy(q_hbm.at[sidx_v.at[c]], rows_v.at[b],
                                  gsems[b]).wait()

        def scatter(c, b):
            pltpu.async_copy(rows_v.at[b], acc.at[didx_v.at[c]], ssems[b],
                             add=True)

        def wait_scatter(c, b):
            pltpu.make_async_copy(rows_v.at[b], acc.at[didx_v.at[c]],
                                  ssems[b]).wait()

        gather(0, 0)
        gather(1, 1)
        wait_gather(0, 0)
        scatter(0, 0)
        wait_gather(1, 1)
        scatter(1, 1)
        gather(2, 2)
        gather(3, 3)

        nquads = (n_chunks - 4) // NB
        c_tail = 2 + NB * nquads

        def quad(o, _):
            for j in range(NB):
                c = NB * o + 2 + j
                b = (2 + j) % NB
                wait_gather(c, b)
                scatter(c, b)
                wait_scatter(c - 2, j)
                gather(c + 2, j)
            return 0

        lax.fori_loop(0, nquads, quad, 0)
        for c in range(c_tail, n_chunks):
            b = c % NB
            wait_gather(c, b)
            scatter(c, b)
            wait_scatter(c - 2, (c - 2) % NB)
            if c + 2 < n_chunks:
                gather(c + 2, (c + 2) % NB)
        wait_scatter(n_chunks - 2, (n_chunks - 2) % NB)
        wait_scatter(n_chunks - 1, (n_chunks - 1) % NB)
        plsc.subcore_barrier()

        pltpu.sync_copy(acc.at[pl.ds(row0, ROWS_PER_SUB)],
                        out_hbm.at[cid, pl.ds(row0, ROWS_PER_SUB)])

    return k(q, src3, dst3)


_BLK = 2000


def _norms_from(dp):
    deg_out = dp[0, 0, :, 0] + dp[1, 0, :, 0]
    deg_in = dp[0, 1, :, 0] + dp[1, 1, :, 0]
    ns = lax.rsqrt(jnp.maximum(deg_out, 1.0))
    nd = lax.rsqrt(jnp.maximum(deg_in, 1.0))
    return ns, nd


def _tc_prep_body(x_ref, dp_ref, o_ref):
    ns, _ = _norms_from(dp_ref[...])
    o_ref[...] = x_ref[...] * ns[:, None]


def _tc_prep(x, degp):
    return pl.pallas_call(
        _tc_prep_body,
        out_shape=jax.ShapeDtypeStruct((N_NODES, D), jnp.float32),
        grid=(N_NODES // _BLK,),
        in_specs=[
            pl.BlockSpec((_BLK, D), lambda i: (i, 0)),
            pl.BlockSpec((NC, 2, _BLK, LANES), lambda i: (0, 0, i, 0)),
        ],
        out_specs=pl.BlockSpec((_BLK, D), lambda i: (i, 0)),
    )(x, degp)


def _tc_layer_body(last, p_ref, dp_ref, w_ref, b_ref, o_ref):
    ns, nd = _norms_from(dp_ref[...])
    s = (p_ref[0] + p_ref[1]) * nd[:, None]
    h = jnp.dot(s, w_ref[...], preferred_element_type=jnp.float32)
    h = jnp.maximum(h + b_ref[...], 0.0)
    if not last:
        h = h * ns[:, None]
    o_ref[...] = h


def _tc_layer(p, degp, w, b2d, last):
    return pl.pallas_call(
        functools.partial(_tc_layer_body, last),
        out_shape=jax.ShapeDtypeStruct((N_NODES, D), jnp.float32),
        grid=(N_NODES // _BLK,),
        in_specs=[
            pl.BlockSpec((NC, _BLK, D), lambda i: (0, i, 0)),
            pl.BlockSpec((NC, 2, _BLK, LANES), lambda i: (0, 0, i, 0)),
            pl.BlockSpec((D, D), lambda i: (0, 0)),
            pl.BlockSpec((1, D), lambda i: (0, 0)),
        ],
        out_specs=pl.BlockSpec((_BLK, D), lambda i: (i, 0)),
    )(p, degp, w, b2d)


def kernel(x, edge_index, W0, b0, W1, b1, W2, b2, W3, b3, W4, b4):
    ei = edge_index.astype(jnp.int32)
    src3 = ei[0].reshape(NW, -1, CHUNK)
    dst3 = ei[1].reshape(NW, -1, CHUNK)
    degp = _sc_degrees(src3, dst3)
    q = _tc_prep(x, degp)
    Ws = [W0, W1, W2, W3, W4]
    bs = [b0, b1, b2, b3, b4]
    for l in range(5):
        p = _sc_propagate(q, src3, dst3)
        q = _tc_layer(p, degp, Ws[l], bs[l].reshape(1, D), last=(l == 4))
    return q


# speedup vs baseline: 12.8687x; 1.2307x over previous
import functools

import jax
import jax.numpy as jnp
from jax import lax
from jax.experimental import pallas as pl
from jax.experimental.pallas import tpu as pltpu
from jax.experimental.pallas import tpu_sc as plsc

N_NODES = 10000
D = 128
NC = 2
NS = 16
LANES = 16
NW = NC * NS

CHUNK = 40
NPAD = 10240
ROWS_PER_SUB = NPAD // NS
NB = 5

_mesh = plsc.VectorSubcoreMesh(core_axis_name="c", subcore_axis_name="s")
_sc_params = pltpu.CompilerParams(use_tc_tiling_on_sc=False)


def _sc_degrees(src3, dst3):
    n_chunks = src3.shape[1]

    @functools.partial(
        pl.kernel,
        out_type=jax.ShapeDtypeStruct((NC, 2, NPAD, LANES), jnp.float32),
        mesh=_mesh,
        scratch_types=[
            pltpu.VMEM((n_chunks, CHUNK), jnp.int32),
            pltpu.VMEM((n_chunks, CHUNK), jnp.int32),
            pltpu.VMEM((CHUNK, LANES), jnp.float32),
            pltpu.VMEM((ROWS_PER_SUB, LANES), jnp.float32),
            pltpu.VMEM_SHARED((NPAD, LANES), jnp.float32),
            pltpu.VMEM_SHARED((NPAD, LANES), jnp.float32),
            pltpu.SemaphoreType.DMA,
        ],
        compiler_params=_sc_params,
    )
    def k(src_hbm, dst_hbm, out_hbm, sidx_v, didx_v, ones_v, zero_v,
          acc0, acc1, ssem):
        cid = lax.axis_index("c")
        sid = lax.axis_index("s")
        wid = cid * NS + sid

        def fill_ones(i, _):
            ones_v[i, :] = jnp.full((LANES,), 1.0, jnp.float32)
            return 0

        lax.fori_loop(0, CHUNK, fill_ones, 0)

        def fill_zero(i, _):
            zero_v[i, :] = jnp.zeros((LANES,), jnp.float32)
            return 0

        lax.fori_loop(0, ROWS_PER_SUB, fill_zero, 0)

        pltpu.sync_copy(src_hbm.at[wid], sidx_v)
        pltpu.sync_copy(dst_hbm.at[wid], didx_v)

        row0 = sid * ROWS_PER_SUB
        pltpu.sync_copy(zero_v, acc0.at[pl.ds(row0, ROWS_PER_SUB)])
        pltpu.sync_copy(zero_v, acc1.at[pl.ds(row0, ROWS_PER_SUB)])
        plsc.subcore_barrier()

        def fire(c, _):
            pltpu.async_copy(ones_v, acc0.at[sidx_v.at[c]], ssem, add=True)
            pltpu.async_copy(ones_v, acc1.at[didx_v.at[c]], ssem, add=True)
            return 0

        lax.fori_loop(0, n_chunks, fire, 0)

        def drain(c, _):
            pltpu.make_async_copy(ones_v, acc0.at[sidx_v.at[c]], ssem).wait()
            pltpu.make_async_copy(ones_v, acc1.at[didx_v.at[c]], ssem).wait()
            return 0

        lax.fori_loop(0, n_chunks, drain, 0)
        plsc.subcore_barrier()

        pltpu.sync_copy(acc0.at[pl.ds(row0, ROWS_PER_SUB)],
                        out_hbm.at[cid, 0, pl.ds(row0, ROWS_PER_SUB)])
        pltpu.sync_copy(acc1.at[pl.ds(row0, ROWS_PER_SUB)],
                        out_hbm.at[cid, 1, pl.ds(row0, ROWS_PER_SUB)])

    return k(src3, dst3)


def _sc_propagate(q, src3, dst3):
    n_chunks = src3.shape[1]
    assert n_chunks >= 8

    @functools.partial(
        pl.kernel,
        out_type=jax.ShapeDtypeStruct((NC, NPAD, D), jnp.float32),
        mesh=_mesh,
        scratch_types=[
            pltpu.VMEM((n_chunks, CHUNK), jnp.int32),
            pltpu.VMEM((n_chunks, CHUNK), jnp.int32),
            pltpu.VMEM((NB, CHUNK, D), jnp.float32),
            pltpu.VMEM_SHARED((NPAD, D), jnp.float32),
            [pltpu.SemaphoreType.DMA] * NB,
            [pltpu.SemaphoreType.DMA] * NB,
        ],
        compiler_params=_sc_params,
    )
    def k(q_hbm, src_hbm, dst_hbm, out_hbm, sidx_v, didx_v, rows_v,
          acc, gsems, ssems):
        cid = lax.axis_index("c")
        sid = lax.axis_index("s")
        wid = cid * NS + sid

        def fill_zero(i, _):
            for j in range(D // LANES):
                rows_v[0, i, pl.ds(j * LANES, LANES)] = jnp.zeros(
                    (LANES,), jnp.float32)
            return 0

        lax.fori_loop(0, CHUNK, fill_zero, 0)

        pltpu.sync_copy(src_hbm.at[wid], sidx_v)
        pltpu.sync_copy(dst_hbm.at[wid], didx_v)

        row0 = sid * ROWS_PER_SUB
        for t in range(ROWS_PER_SUB // CHUNK):
            pltpu.sync_copy(rows_v.at[0], acc.at[pl.ds(row0 + t * CHUNK, CHUNK)])
        plsc.subcore_barrier()

        def gather(c, b):
            pltpu.async_copy(q_hbm.at[sidx_v.at[c]], rows_v.at[b], gsems[b])

        def wait_gather(c, b):
            pltpu.make_async_copy(q_hbm.at[sidx_v.at[c]], rows_v.at[b],
                                  gsems[b]).wait()

        def scatter(c, b):
            pltpu.async_copy(rows_v.at[b], acc.at[didx_v.at[c]], ssems[b],
                             add=True)

        def wait_scatter(c, b):
            pltpu.make_async_copy(rows_v.at[b], acc.at[didx_v.at[c]],
                                  ssems[b]).wait()

        gather(0, 0)
        gather(1, 1)
        gather(2, 2)
        wait_gather(0, 0)
        scatter(0, 0)
        gather(3, 3)
        wait_gather(1, 1)
        scatter(1, 1)
        gather(4, 4)

        gd = NB - 2
        ngroups = (n_chunks - 2 - gd) // NB
        c_tail = 2 + NB * ngroups

        def group(o, _):
            for j in range(NB):
                c = NB * o + 2 + j
                b = (2 + j) % NB
                wait_gather(c, b)
                scatter(c, b)
                wait_scatter(c - 2, j)
                gather(c + gd, j)
            return 0

        lax.fori_loop(0, ngroups, group, 0)
        for c in range(c_tail, n_chunks):
            b = c % NB
            wait_gather(c, b)
            scatter(c, b)
            wait_scatter(c - 2, (c - 2) % NB)
            if c + gd < n_chunks:
                gather(c + gd, (c + gd) % NB)
        wait_scatter(n_chunks - 2, (n_chunks - 2) % NB)
        wait_scatter(n_chunks - 1, (n_chunks - 1) % NB)
        plsc.subcore_barrier()

        pltpu.sync_copy(acc.at[pl.ds(row0, ROWS_PER_SUB)],
                        out_hbm.at[cid, pl.ds(row0, ROWS_PER_SUB)])

    return k(q, src3, dst3)


_BLK = 2000


def _norms_from(dp):
    deg_out = dp[0, 0, :, 0] + dp[1, 0, :, 0]
    deg_in = dp[0, 1, :, 0] + dp[1, 1, :, 0]
    ns = lax.rsqrt(jnp.maximum(deg_out, 1.0))
    nd = lax.rsqrt(jnp.maximum(deg_in, 1.0))
    return ns, nd


def _tc_prep_body(x_ref, dp_ref, o_ref):
    ns, _ = _norms_from(dp_ref[...])
    o_ref[...] = x_ref[...] * ns[:, None]


def _tc_prep(x, degp):
    return pl.pallas_call(
        _tc_prep_body,
        out_shape=jax.ShapeDtypeStruct((N_NODES, D), jnp.float32),
        grid=(N_NODES // _BLK,),
        in_specs=[
            pl.BlockSpec((_BLK, D), lambda i: (i, 0)),
            pl.BlockSpec((NC, 2, _BLK, LANES), lambda i: (0, 0, i, 0)),
        ],
        out_specs=pl.BlockSpec((_BLK, D), lambda i: (i, 0)),
    )(x, degp)


def _tc_layer_body(last, p_ref, dp_ref, w_ref, b_ref, o_ref):
    ns, nd = _norms_from(dp_ref[...])
    s = (p_ref[0] + p_ref[1]) * nd[:, None]
    h = jnp.dot(s, w_ref[...], preferred_element_type=jnp.float32)
    h = jnp.maximum(h + b_ref[...], 0.0)
    if not last:
        h = h * ns[:, None]
    o_ref[...] = h


def _tc_layer(p, degp, w, b2d, last):
    return pl.pallas_call(
        functools.partial(_tc_layer_body, last),
        out_shape=jax.ShapeDtypeStruct((N_NODES, D), jnp.float32),
        grid=(N_NODES // _BLK,),
        in_specs=[
            pl.BlockSpec((NC, _BLK, D), lambda i: (0, i, 0)),
            pl.BlockSpec((NC, 2, _BLK, LANES), lambda i: (0, 0, i, 0)),
            pl.BlockSpec((D, D), lambda i: (0, 0)),
            pl.BlockSpec((1, D), lambda i: (0, 0)),
        ],
        out_specs=pl.BlockSpec((_BLK, D), lambda i: (i, 0)),
    )(p, degp, w, b2d)


def kernel(x, edge_index, W0, b0, W1, b1, W2, b2, W3, b3, W4, b4):
    ei = edge_index.astype(jnp.int32)
    src3 = ei[0].reshape(NW, -1, CHUNK)
    dst3 = ei[1].reshape(NW, -1, CHUNK)
    degp = _sc_degrees(src3, dst3)
    q = _tc_prep(x, degp)
    Ws = [W0, W1, W2, W3, W4]
    bs = [b0, b1, b2, b3, b4]
    for l in range(5):
        p = _sc_propagate(q, src3, dst3)
        q = _tc_layer(p, degp, Ws[l], bs[l].reshape(1, D), last=(l == 4))
    return q
